# Initial kernel scaffold; baseline (speedup 1.0000x reference)
import jax
import jax.numpy as jnp
from jax import lax
from jax.experimental import pallas as pl
from jax.experimental.pallas import tpu as pltpu
from jax.experimental.pallas import tpu_sc as plsc

_NC = 2
_NS = 16
_NW = _NC * _NS
_CHUNK = 128


def _gather_body(nrows, x_hbm, tab_hbm, out_hbm, idx_v, rows_v, sem):
    wid = lax.axis_index("s") * _NC + lax.axis_index("c")
    base = wid * nrows
    pltpu.sync_copy(x_hbm.at[pl.ds(base, nrows)], idx_v)

    def chunk(j, carry):
        pltpu.async_copy(tab_hbm.at[idx_v.at[j]], rows_v, sem).wait()
        pltpu.sync_copy(rows_v, out_hbm.at[pl.ds((base + j) * _CHUNK, _CHUNK)])
        return carry

    lax.fori_loop(0, nrows, chunk, 0)


def kernel(x, weight):
    B, T = x.shape
    V, D = weight.shape
    n = B * T
    assert n % (_NW * _CHUNK) == 0
    nrows = n // (_NW * _CHUNK)

    x2 = x.reshape(n // _CHUNK, _CHUNK).astype(jnp.int32)
    mesh = plsc.VectorSubcoreMesh(core_axis_name="c", subcore_axis_name="s")

    body = lambda *refs: _gather_body(nrows, *refs)
    out = pl.kernel(
        body,
        out_type=jax.ShapeDtypeStruct((n, D), jnp.float32),
        mesh=mesh,
        scratch_types=[
            pltpu.VMEM((nrows, _CHUNK), jnp.int32),
            pltpu.VMEM((_CHUNK, D), jnp.float32),
            pltpu.SemaphoreType.DMA,
        ],
    )(x2, weight)
    return out.reshape(B, T, D)


# speedup vs baseline: 2.9638x; 2.9638x over previous
import jax
import jax.numpy as jnp
from jax import lax
from jax.experimental import pallas as pl
from jax.experimental.pallas import tpu as pltpu
from jax.experimental.pallas import tpu_sc as plsc

_NC = 2
_NS = 16
_NW = _NC * _NS
_CHUNK = 128


def _gather_body(nrows, x_hbm, tab_hbm, out_hbm, idx_v, rows_v, sem):
    wid = lax.axis_index("s") * _NC + lax.axis_index("c")
    base = wid * nrows
    pltpu.sync_copy(x_hbm.at[wid], idx_v)

    def chunk(j, carry):
        pltpu.async_copy(tab_hbm.at[idx_v.at[j]], rows_v, sem).wait()
        off = pl.multiple_of((base + j) * _CHUNK, 8)
        pltpu.sync_copy(rows_v, out_hbm.at[pl.ds(off, _CHUNK)])
        return carry

    lax.fori_loop(0, nrows, chunk, 0)


def kernel(x, weight):
    B, T = x.shape
    V, D = weight.shape
    n = B * T
    assert n % (_NW * _CHUNK) == 0
    nrows = n // (_NW * _CHUNK)

    x2 = x.reshape(_NW, nrows, _CHUNK).astype(jnp.int32)
    mesh = plsc.VectorSubcoreMesh(core_axis_name="c", subcore_axis_name="s")

    body = lambda *refs: _gather_body(nrows, *refs)
    out = pl.kernel(
        body,
        out_type=jax.ShapeDtypeStruct((n, D), jnp.float32),
        mesh=mesh,
        scratch_types=[
            pltpu.VMEM((nrows, _CHUNK), jnp.int32),
            pltpu.VMEM((_CHUNK, D), jnp.float32),
            pltpu.SemaphoreType.DMA,
        ],
    )(x2, weight)
    return out.reshape(B, T, D)


# speedup vs baseline: 3.3125x; 1.1177x over previous
import jax
import jax.numpy as jnp
from jax import lax
from jax.experimental import pallas as pl
from jax.experimental.pallas import tpu as pltpu
from jax.experimental.pallas import tpu_sc as plsc

_NC = 2
_NS = 16
_NW = _NC * _NS
_CHUNK = 128
_NBUF = 5


def _gather_body(nrows, x_hbm, tab_hbm, out_hbm, idx_v, *rest):
    bufs = rest[:_NBUF]
    gsems = rest[_NBUF:2 * _NBUF]
    wsems = rest[2 * _NBUF:3 * _NBUF]
    wid = lax.axis_index("s") * _NC + lax.axis_index("c")
    base = wid * nrows
    pltpu.sync_copy(x_hbm.at[wid], idx_v)

    def out_slice(j):
        off = pl.multiple_of((base + j) * _CHUNK, 8)
        return out_hbm.at[pl.ds(off, _CHUNK)]

    for b in range(_NBUF):
        pltpu.async_copy(tab_hbm.at[idx_v.at[b]], bufs[b], gsems[b])

    def outer(t, carry):
        for b in range(_NBUF):
            j = t * _NBUF + b
            pltpu.make_async_copy(tab_hbm.at[idx_v.at[j]], bufs[b],
                                  gsems[b]).wait()
            pltpu.async_copy(bufs[b], out_slice(j), wsems[b])
        for b in range(_NBUF):
            jn = (t + 1) * _NBUF + b

            @pl.when(jn < nrows)
            def _():
                pltpu.make_async_copy(bufs[b], out_slice(jn), wsems[b]).wait()
                pltpu.async_copy(tab_hbm.at[idx_v.at[jn]], bufs[b], gsems[b])

        return carry

    lax.fori_loop(0, nrows // _NBUF, outer, 0)

    for b in range(_NBUF):
        pltpu.make_async_copy(bufs[b], out_slice(0), wsems[b]).wait()


def kernel(x, weight):
    B, T = x.shape
    V, D = weight.shape
    n = B * T
    assert n % (_NW * _CHUNK) == 0
    nrows = n // (_NW * _CHUNK)
    assert nrows % _NBUF == 0

    x2 = x.reshape(_NW, nrows, _CHUNK).astype(jnp.int32)
    mesh = plsc.VectorSubcoreMesh(core_axis_name="c", subcore_axis_name="s")

    body = lambda *refs: _gather_body(nrows, *refs)
    out = pl.kernel(
        body,
        out_type=jax.ShapeDtypeStruct((n, D), jnp.float32),
        mesh=mesh,
        scratch_types=(
            [pltpu.VMEM((nrows, _CHUNK), jnp.int32)]
            + [pltpu.VMEM((_CHUNK, D), jnp.float32) for _ in range(_NBUF)]
            + [pltpu.SemaphoreType.DMA for _ in range(2 * _NBUF)]
        ),
    )(x2, weight)
    return out.reshape(B, T, D)
